# baseline (speedup 1.0000x reference)
import functools

import jax
import jax.numpy as jnp
from jax import lax
from jax.experimental import pallas as pl
from jax.experimental.pallas import tpu as pltpu
from jax.experimental.pallas import tpu_sc as plsc

N = 50000
NP = 50176
E = 800000
EP = 819200
NG = 64
NC = 2
NS = 16
NW = NC * NS
EW = EP // NW
KW = 4
BW = 128 * KW
NWIN = EW // BW
ER = EP // 128
BN_BLK = 1024
GRID_N = NP // BN_BLK
BE_BLK = 2048
GRID_E = EP // BE_BLK
ZB = 64
NPS = NP // NS
ZW = NPS // ZB
NPW = NP // NW
BP = 112
NPWIN = NPW // BP
F32 = jnp.float32
I32 = jnp.int32

_SC_PARAMS = pltpu.CompilerParams(
    needs_layout_passes=False, use_tc_tiling_on_sc=False)


def _elu(x):
    return jnp.where(x > 0, x, jnp.exp(jnp.where(x > 0, 0.0, x)) - 1.0)


def _basis_body(e_ref, o_ref):
    f0 = e_ref[:, 0:1]
    f1 = e_ref[:, 1:2]
    o_ref[...] = jnp.concatenate(
        [(1.0 - f0) * (1.0 - f1), f0 * (1.0 - f1), (1.0 - f0) * f1, f0 * f1,
         jnp.zeros((BE_BLK, 4), F32)], axis=1)


def _basis(eap):
    return pl.pallas_call(
        _basis_body,
        grid=(GRID_E,),
        in_specs=[pl.BlockSpec((BE_BLK, 8), lambda i: (i, 0))],
        out_specs=pl.BlockSpec((BE_BLK, 8), lambda i: (i, 0)),
        out_shape=jax.ShapeDtypeStruct((EP, 8), F32),
    )(eap)


def _msg_body(cout, with_count, g_ref, bas_ref, o_ref):
    g = g_ref[...]
    bas = bas_ref[...]
    msg = bas[:, 0:1] * g[:, 0:cout]
    for kk in range(1, 4):
        msg = msg + bas[:, kk:kk + 1] * g[:, kk * cout:(kk + 1) * cout]
    if with_count:
        msg = jnp.concatenate(
            [msg, jnp.ones((BE_BLK, 1), F32), jnp.zeros((BE_BLK, 7), F32)],
            axis=1)
    o_ref[...] = msg


def _msg(g, bas_em, cout, cacc, with_count):
    r = 4 * cout
    return pl.pallas_call(
        functools.partial(_msg_body, cout, with_count),
        grid=(GRID_E,),
        in_specs=[
            pl.BlockSpec((BE_BLK, r), lambda i: (i, 0)),
            pl.BlockSpec((BE_BLK, 8), lambda i: (i, 0)),
        ],
        out_specs=pl.BlockSpec((BE_BLK, cacc), lambda i: (i, 0)),
        out_shape=jax.ShapeDtypeStruct((EP, cacc), F32),
    )(g, bas_em)


def _mm_body(x_ref, w_ref, o_ref):
    o_ref[...] = jnp.dot(x_ref[...], w_ref[...], preferred_element_type=F32)


def _mm_plain(xp, wc):
    cin, r = wc.shape
    return pl.pallas_call(
        _mm_body,
        grid=(GRID_N,),
        in_specs=[
            pl.BlockSpec((BN_BLK, cin), lambda i: (i, 0)),
            pl.BlockSpec((cin, r), lambda i: (0, 0)),
        ],
        out_specs=pl.BlockSpec((BN_BLK, r), lambda i: (i, 0)),
        out_shape=jax.ShapeDtypeStruct((NP, r), F32),
    )(xp, wc)


def _mm_bn_body(t_ref, s1_ref, s2_ref, g_ref, b_ref, w_ref, o_ref):
    mu = s1_ref[...] / N
    var = s2_ref[...] / N - mu * mu
    a = g_ref[...] * lax.rsqrt(var + 1e-5)
    c = b_ref[...] - mu * a
    o_ref[...] = jnp.dot(t_ref[...] * a + c, w_ref[...],
                         preferred_element_type=F32)


def _mm_bn(t, s1, s2, g, b, wc):
    cin, r = wc.shape
    return pl.pallas_call(
        _mm_bn_body,
        grid=(GRID_N,),
        in_specs=[
            pl.BlockSpec((BN_BLK, cin), lambda i: (i, 0)),
            pl.BlockSpec((1, cin), lambda i: (0, 0)),
            pl.BlockSpec((1, cin), lambda i: (0, 0)),
            pl.BlockSpec((1, cin), lambda i: (0, 0)),
            pl.BlockSpec((1, cin), lambda i: (0, 0)),
            pl.BlockSpec((cin, r), lambda i: (0, 0)),
        ],
        out_specs=pl.BlockSpec((BN_BLK, r), lambda i: (i, 0)),
        out_shape=jax.ShapeDtypeStruct((NP, r), F32),
    )(t, s1, s2, g, b, wc)


def _fin1_body(acc_ref, t_ref, inv_ref, s1_ref, s2_ref):
    i = pl.program_id(0)
    a = acc_ref[0] + acc_ref[1]
    cnt = a[:, 8:9]
    inv = 1.0 / jnp.maximum(cnt, 1.0)
    rows = i * BN_BLK + lax.broadcasted_iota(I32, (BN_BLK, 16), 0)
    cols = lax.broadcasted_iota(I32, (BN_BLK, 16), 1)
    t = _elu(a * inv)
    t = jnp.where((rows < N) & (cols < 8), t, 0.0)
    t_ref[...] = t
    inv_ref[...] = jnp.broadcast_to(inv, (BN_BLK, 16))

    @pl.when(i == 0)
    def _():
        s1_ref[...] = jnp.zeros_like(s1_ref)
        s2_ref[...] = jnp.zeros_like(s2_ref)

    s1_ref[...] += jnp.sum(t, axis=0, keepdims=True)
    s2_ref[...] += jnp.sum(t * t, axis=0, keepdims=True)


def _fin1(acc):
    return pl.pallas_call(
        _fin1_body,
        grid=(GRID_N,),
        in_specs=[pl.BlockSpec((2, BN_BLK, 16), lambda i: (0, i, 0))],
        out_specs=[
            pl.BlockSpec((BN_BLK, 16), lambda i: (i, 0)),
            pl.BlockSpec((BN_BLK, 16), lambda i: (i, 0)),
            pl.BlockSpec((1, 16), lambda i: (0, 0)),
            pl.BlockSpec((1, 16), lambda i: (0, 0)),
        ],
        out_shape=[
            jax.ShapeDtypeStruct((NP, 16), F32),
            jax.ShapeDtypeStruct((NP, 16), F32),
            jax.ShapeDtypeStruct((1, 16), F32),
            jax.ShapeDtypeStruct((1, 16), F32),
        ],
    )(acc)


def _fin23_body(c_dim, acc_ref, inv_ref, t_ref, s1_ref, s2_ref):
    i = pl.program_id(0)
    a = acc_ref[0] + acc_ref[1]
    inv = inv_ref[:, 0:1]
    rows = i * BN_BLK + lax.broadcasted_iota(I32, (BN_BLK, c_dim), 0)
    t = _elu(a * inv)
    t = jnp.where(rows < N, t, 0.0)
    t_ref[...] = t

    @pl.when(i == 0)
    def _():
        s1_ref[...] = jnp.zeros_like(s1_ref)
        s2_ref[...] = jnp.zeros_like(s2_ref)

    s1_ref[...] += jnp.sum(t, axis=0, keepdims=True)
    s2_ref[...] += jnp.sum(t * t, axis=0, keepdims=True)


def _fin23(acc, inv16, c_dim):
    return pl.pallas_call(
        functools.partial(_fin23_body, c_dim),
        grid=(GRID_N,),
        in_specs=[
            pl.BlockSpec((2, BN_BLK, c_dim), lambda i: (0, i, 0)),
            pl.BlockSpec((BN_BLK, 16), lambda i: (i, 0)),
        ],
        out_specs=[
            pl.BlockSpec((BN_BLK, c_dim), lambda i: (i, 0)),
            pl.BlockSpec((1, c_dim), lambda i: (0, 0)),
            pl.BlockSpec((1, c_dim), lambda i: (0, 0)),
        ],
        out_shape=[
            jax.ShapeDtypeStruct((NP, c_dim), F32),
            jax.ShapeDtypeStruct((1, c_dim), F32),
            jax.ShapeDtypeStruct((1, c_dim), F32),
        ],
    )(acc, inv16)


def _head_body(pa_ref, pc_ref, s1_ref, s2_ref, g_ref, b_ref, w_ref, o_ref):
    ps = (pa_ref[0] + pa_ref[1])[0:NG, :]
    cnt = (pc_ref[0] + pc_ref[1])[0:NG, 0:1]
    pooled = ps / jnp.maximum(cnt, 1.0)
    mu = s1_ref[...] / N
    var = s2_ref[...] / N - mu * mu
    a = g_ref[...] * lax.rsqrt(var + 1e-5)
    c = b_ref[...] - mu * a
    o_ref[...] = jnp.dot(pooled * a + c, w_ref[...],
                         preferred_element_type=F32)


def _head(pa, pc, s1, s2, g, b, fcwp):
    return pl.pallas_call(
        _head_body,
        out_shape=jax.ShapeDtypeStruct((NG, 128), F32),
    )(pa, pc, s1, s2, g, b, fcwp)


def _sc_gather(y, srcp2d, cout):
    r = 4 * cout
    mesh = plsc.VectorSubcoreMesh(core_axis_name="c", subcore_axis_name="s")

    @functools.partial(
        pl.kernel,
        out_type=jax.ShapeDtypeStruct((EP, r), F32),
        mesh=mesh,
        compiler_params=_SC_PARAMS,
        scratch_types=[
            pltpu.VMEM((KW, 128), I32),
            pltpu.VMEM((BW, r), F32),
            pltpu.SemaphoreType.DMA,
        ],
    )
    def k(y_hbm, src_hbm, g_hbm, idx_v, rows_v, sem):
        cid = lax.axis_index("c")
        sid = lax.axis_index("s")
        wid = sid * NC + cid
        row_base = wid * (EW // 128)

        def wbody(w, carry):
            r0 = row_base + w * KW
            pltpu.sync_copy(src_hbm.at[pl.ds(r0, KW), :], idx_v)
            descs = []
            for j in range(KW):
                descs.append(pltpu.async_copy(
                    y_hbm.at[idx_v.at[j]],
                    rows_v.at[pl.ds(j * 128, 128), :], sem))
            for d in descs:
                d.wait()
            pltpu.sync_copy(rows_v, g_hbm.at[pl.ds(r0 * 128, BW), :])
            return carry
        lax.fori_loop(0, NWIN, wbody, 0)

    return k(y, srcp2d)


def _sc_scatter(m, dstp2d, cacc):
    mesh = plsc.VectorSubcoreMesh(core_axis_name="c", subcore_axis_name="s")

    @functools.partial(
        pl.kernel,
        out_type=jax.ShapeDtypeStruct((NC, NP, cacc), F32),
        mesh=mesh,
        compiler_params=_SC_PARAMS,
        scratch_types=[
            pltpu.VMEM((KW, 128), I32),
            pltpu.VMEM((BW, cacc), F32),
            pltpu.VMEM((ZB, cacc), F32),
            pltpu.VMEM_SHARED((NP, cacc), F32),
        ],
    )
    def k(m_hbm, dst_hbm, out_hbm, idx_v, msg_v, zb_v, acc_sh):
        cid = lax.axis_index("c")
        sid = lax.axis_index("s")
        wid = sid * NC + cid
        zvec = jnp.zeros((16,), F32)

        for i in range(ZB):
            for c0 in range(0, cacc, 16):
                zb_v[i, pl.ds(c0, 16)] = zvec
        a0 = sid * NPS

        def zbody(w, carry):
            pltpu.sync_copy(zb_v, acc_sh.at[pl.ds(a0 + w * ZB, ZB), :])
            return carry
        lax.fori_loop(0, ZW, zbody, 0)
        plsc.subcore_barrier()

        row_base = wid * (EW // 128)

        def wbody(w, carry):
            r0 = row_base + w * KW
            pltpu.sync_copy(dst_hbm.at[pl.ds(r0, KW), :], idx_v)
            pltpu.sync_copy(m_hbm.at[pl.ds(r0 * 128, BW), :], msg_v)
            for j in range(KW):
                pltpu.sync_copy(msg_v.at[pl.ds(j * 128, 128), :],
                                acc_sh.at[idx_v.at[j]], add=True)
            return carry
        lax.fori_loop(0, NWIN, wbody, 0)
        plsc.subcore_barrier()

        def obody(w, carry):
            sl = pl.ds(a0 + w * ZB, ZB)
            pltpu.sync_copy(acc_sh.at[sl, :], out_hbm.at[cid, sl, :])
            return carry
        lax.fori_loop(0, ZW, obody, 0)

    return k(m, dstp2d)


def _sc_pool(t3, bidp):
    mesh = plsc.VectorSubcoreMesh(core_axis_name="c", subcore_axis_name="s")

    @functools.partial(
        pl.kernel,
        out_type=(
            jax.ShapeDtypeStruct((NC, NG + 1, 32), F32),
            jax.ShapeDtypeStruct((NC, NG + 1, 16), F32),
        ),
        mesh=mesh,
        compiler_params=_SC_PARAMS,
        scratch_types=[
            pltpu.VMEM((BP, 32), F32),
            pltpu.VMEM((BP, 16), F32),
            pltpu.VMEM((BP,), I32),
            pltpu.VMEM_SHARED((NG + 1, 32), F32),
            pltpu.VMEM_SHARED((NG + 1, 16), F32),
        ],
    )
    def k(t3_hbm, bid_hbm, pa_hbm, pc_hbm,
          tw_v, ones_v, bid_v, pacc_sh, pcnt_sh):
        cid = lax.axis_index("c")
        sid = lax.axis_index("s")
        wid = sid * NC + cid
        iot = lax.iota(I32, 16)
        zvec = jnp.zeros((16,), F32)
        zrow = jnp.zeros((16,), I32)
        ovec = jnp.ones((16,), F32)

        def zb(i, carry):
            plsc.store_scatter(tw_v, [zrow + i, iot], zvec)
            plsc.store_scatter(tw_v, [zrow + i, iot + 16], zvec)
            plsc.store_scatter(ones_v, [zrow + i, iot], ovec)
            return carry
        lax.fori_loop(0, BP, zb, 0)

        @pl.when(sid == 0)
        def _():
            pltpu.sync_copy(tw_v.at[pl.ds(0, NG + 1), :], pacc_sh)
            pltpu.sync_copy(tw_v.at[pl.ds(0, NG + 1), 0:16], pcnt_sh)
        plsc.subcore_barrier()

        n_base = wid * NPW

        def wbody(w, carry):
            n0 = n_base + w * BP
            pltpu.sync_copy(t3_hbm.at[pl.ds(n0, BP), :], tw_v)
            pltpu.sync_copy(bid_hbm.at[pl.ds(n0, BP)], bid_v)
            pltpu.sync_copy(tw_v, pacc_sh.at[bid_v], add=True)
            pltpu.sync_copy(ones_v, pcnt_sh.at[bid_v], add=True)
            return carry
        lax.fori_loop(0, NPWIN, wbody, 0)
        plsc.subcore_barrier()

        @pl.when(sid == 0)
        def _():
            pltpu.sync_copy(pacc_sh, pa_hbm.at[cid])
            pltpu.sync_copy(pcnt_sh, pc_hbm.at[cid])

    return k(t3, bidp)


def _wcat(w):
    return jnp.transpose(w, (1, 0, 2)).reshape(w.shape[1], 4 * w.shape[2])


def _layer(y, srcp2d, dstp2d, bas_em, cout, cacc, with_count):
    g = _sc_gather(y, srcp2d, cout)
    m = _msg(g, bas_em, cout, cacc, with_count)
    return _sc_scatter(m, dstp2d, cacc)


@jax.jit
def kernel(x, edge_index, edge_attr, batch,
           W1, g1, b1, W2, g2, b2, W3, g3, b3, fcW):
    src = edge_index[0]
    dst = edge_index[1]
    srcp2d = jnp.pad(src, (0, EP - E)).reshape(ER, 128)
    dstp2d = jnp.pad(dst, (0, EP - E),
                     constant_values=N).reshape(ER, 128)
    eap = jnp.pad(edge_attr, ((0, EP - E), (0, 6)))
    xp = jnp.pad(x, ((0, NP - N), (0, 5)))
    bidp = jnp.pad(batch, (0, NP - N), constant_values=NG)

    w1c = jnp.pad(_wcat(W1), ((0, 5), (0, 0)))
    w2c = jnp.pad(_wcat(W2), ((0, 8), (0, 0)))
    w3c = _wcat(W3)
    g1p = jnp.pad(g1, (0, 8)).reshape(1, 16)
    b1p = jnp.pad(b1, (0, 8)).reshape(1, 16)
    g2r = g2.reshape(1, 16)
    b2r = b2.reshape(1, 16)
    g3r = g3.reshape(1, 32)
    b3r = b3.reshape(1, 32)
    fcwp = jnp.pad(fcW, ((0, 0), (0, 118)))

    bas_em = _basis(eap)
    y1 = _mm_plain(xp, w1c)
    acc1 = _layer(y1, srcp2d, dstp2d, bas_em, cout=8, cacc=16,
                  with_count=True)
    t1, inv16, s1a, s2a = _fin1(acc1)
    y2 = _mm_bn(t1, s1a, s2a, g1p, b1p, w2c)
    acc2 = _layer(y2, srcp2d, dstp2d, bas_em, cout=16, cacc=16,
                  with_count=False)
    t2, s1b, s2b = _fin23(acc2, inv16, 16)
    y3 = _mm_bn(t2, s1b, s2b, g2r, b2r, w3c)
    acc3 = _layer(y3, srcp2d, dstp2d, bas_em, cout=32, cacc=32,
                  with_count=False)
    t3, s1c, s2c = _fin23(acc3, inv16, 32)
    pa, pc = _sc_pool(t3, bidp)
    out = _head(pa, pc, s1c, s2c, g3r, b3r, fcwp)
    return out[:, :10]


# speedup vs baseline: 1.3124x; 1.3124x over previous
import functools

import jax
import jax.numpy as jnp
from jax import lax
from jax.experimental import pallas as pl
from jax.experimental.pallas import tpu as pltpu
from jax.experimental.pallas import tpu_sc as plsc

N = 50000
NP = 50176
E = 800000
EP = 819200
NG = 64
NC = 2
NS = 16
NW = NC * NS
EW = EP // NW
BW = 128
NWIN = EW // BW
BN_BLK = 1024
GRID_N = NP // BN_BLK
BE_BLK = 6400
GRID_E = EP // BE_BLK
ZB = 32
NPS = NP // NS
ZW = NPS // ZB
NPW = NP // NW
BP = 112
NPWIN = NPW // BP
F32 = jnp.float32
I32 = jnp.int32


def _elu(x):
    return jnp.where(x > 0, x, jnp.exp(jnp.where(x > 0, 0.0, x)) - 1.0)


def _basis_body(e_ref, o_ref):
    f0 = e_ref[0:1, :]
    f1 = e_ref[1:2, :]
    o_ref[0:1, :] = (1.0 - f0) * (1.0 - f1)
    o_ref[1:2, :] = f0 * (1.0 - f1)
    o_ref[2:3, :] = (1.0 - f0) * f1
    o_ref[3:4, :] = f0 * f1
    o_ref[4:8, :] = jnp.zeros((4, BE_BLK), F32)


def _basis(eat):
    return pl.pallas_call(
        _basis_body,
        grid=(GRID_E,),
        in_specs=[pl.BlockSpec((8, BE_BLK), lambda i: (0, i))],
        out_specs=pl.BlockSpec((8, BE_BLK), lambda i: (0, i)),
        out_shape=jax.ShapeDtypeStruct((8, EP), F32),
    )(eat)


BE2 = 2048
GRID_E2 = EP // BE2


def _bex_body(e_ref, o_ref):
    f0 = e_ref[:, 0:1]
    f1 = e_ref[:, 1:2]
    parts = []
    for b in [(1.0 - f0) * (1.0 - f1), f0 * (1.0 - f1),
              (1.0 - f0) * f1, f0 * f1]:
        parts.append(jnp.broadcast_to(b, (BE2, 16)))
    o_ref[...] = jnp.concatenate(parts, axis=1)


def _bex(eap):
    return pl.pallas_call(
        _bex_body,
        grid=(GRID_E2,),
        in_specs=[pl.BlockSpec((BE2, 8), lambda i: (i, 0))],
        out_specs=pl.BlockSpec((BE2, 64), lambda i: (i, 0)),
        out_shape=jax.ShapeDtypeStruct((EP, 64), F32),
    )(eap)


def _mm_body(x_ref, w_ref, o_ref):
    o_ref[...] = jnp.dot(x_ref[...], w_ref[...], preferred_element_type=F32)


def _mm_plain(xp, wc):
    cin, r = wc.shape
    return pl.pallas_call(
        _mm_body,
        grid=(GRID_N,),
        in_specs=[
            pl.BlockSpec((BN_BLK, cin), lambda i: (i, 0)),
            pl.BlockSpec((cin, r), lambda i: (0, 0)),
        ],
        out_specs=pl.BlockSpec((BN_BLK, r), lambda i: (i, 0)),
        out_shape=jax.ShapeDtypeStruct((NP, r), F32),
    )(xp, wc)


def _mm_bn_body(t_ref, s1_ref, s2_ref, g_ref, b_ref, w_ref, o_ref):
    mu = s1_ref[...] / N
    var = s2_ref[...] / N - mu * mu
    a = g_ref[...] * lax.rsqrt(var + 1e-5)
    c = b_ref[...] - mu * a
    o_ref[...] = jnp.dot(t_ref[...] * a + c, w_ref[...],
                         preferred_element_type=F32)


def _mm_bn(t, s1, s2, g, b, wc):
    cin, r = wc.shape
    return pl.pallas_call(
        _mm_bn_body,
        grid=(GRID_N,),
        in_specs=[
            pl.BlockSpec((BN_BLK, cin), lambda i: (i, 0)),
            pl.BlockSpec((1, cin), lambda i: (0, 0)),
            pl.BlockSpec((1, cin), lambda i: (0, 0)),
            pl.BlockSpec((1, cin), lambda i: (0, 0)),
            pl.BlockSpec((1, cin), lambda i: (0, 0)),
            pl.BlockSpec((cin, r), lambda i: (0, 0)),
        ],
        out_specs=pl.BlockSpec((BN_BLK, r), lambda i: (i, 0)),
        out_shape=jax.ShapeDtypeStruct((NP, r), F32),
    )(t, s1, s2, g, b, wc)


def _fin1_body(acc_ref, t_ref, inv_ref, s1_ref, s2_ref):
    i = pl.program_id(0)
    a = acc_ref[0] + acc_ref[1]
    cnt = a[:, 8:9]
    inv = 1.0 / jnp.maximum(cnt, 1.0)
    rows = i * BN_BLK + lax.broadcasted_iota(I32, (BN_BLK, 16), 0)
    cols = lax.broadcasted_iota(I32, (BN_BLK, 16), 1)
    t = _elu(a * inv)
    t = jnp.where((rows < N) & (cols < 8), t, 0.0)
    t_ref[...] = t
    inv_ref[...] = jnp.broadcast_to(inv, (BN_BLK, 16))

    @pl.when(i == 0)
    def _():
        s1_ref[...] = jnp.zeros_like(s1_ref)
        s2_ref[...] = jnp.zeros_like(s2_ref)

    s1_ref[...] += jnp.sum(t, axis=0, keepdims=True)
    s2_ref[...] += jnp.sum(t * t, axis=0, keepdims=True)


def _fin1(acc):
    return pl.pallas_call(
        _fin1_body,
        grid=(GRID_N,),
        in_specs=[pl.BlockSpec((2, BN_BLK, 16), lambda i: (0, i, 0))],
        out_specs=[
            pl.BlockSpec((BN_BLK, 16), lambda i: (i, 0)),
            pl.BlockSpec((BN_BLK, 16), lambda i: (i, 0)),
            pl.BlockSpec((1, 16), lambda i: (0, 0)),
            pl.BlockSpec((1, 16), lambda i: (0, 0)),
        ],
        out_shape=[
            jax.ShapeDtypeStruct((NP, 16), F32),
            jax.ShapeDtypeStruct((NP, 16), F32),
            jax.ShapeDtypeStruct((1, 16), F32),
            jax.ShapeDtypeStruct((1, 16), F32),
        ],
    )(acc)


def _fin23_body(c_dim, acc_ref, inv_ref, t_ref, s1_ref, s2_ref):
    i = pl.program_id(0)
    a = acc_ref[0] + acc_ref[1]
    inv = inv_ref[:, 0:1]
    rows = i * BN_BLK + lax.broadcasted_iota(I32, (BN_BLK, c_dim), 0)
    t = _elu(a * inv)
    t = jnp.where(rows < N, t, 0.0)
    t_ref[...] = t

    @pl.when(i == 0)
    def _():
        s1_ref[...] = jnp.zeros_like(s1_ref)
        s2_ref[...] = jnp.zeros_like(s2_ref)

    s1_ref[...] += jnp.sum(t, axis=0, keepdims=True)
    s2_ref[...] += jnp.sum(t * t, axis=0, keepdims=True)


def _fin23(acc, inv16, c_dim):
    return pl.pallas_call(
        functools.partial(_fin23_body, c_dim),
        grid=(GRID_N,),
        in_specs=[
            pl.BlockSpec((2, BN_BLK, c_dim), lambda i: (0, i, 0)),
            pl.BlockSpec((BN_BLK, 16), lambda i: (i, 0)),
        ],
        out_specs=[
            pl.BlockSpec((BN_BLK, c_dim), lambda i: (i, 0)),
            pl.BlockSpec((1, c_dim), lambda i: (0, 0)),
            pl.BlockSpec((1, c_dim), lambda i: (0, 0)),
        ],
        out_shape=[
            jax.ShapeDtypeStruct((NP, c_dim), F32),
            jax.ShapeDtypeStruct((1, c_dim), F32),
            jax.ShapeDtypeStruct((1, c_dim), F32),
        ],
    )(acc, inv16)


def _head_body(pa_ref, pc_ref, s1_ref, s2_ref, g_ref, b_ref, w_ref, o_ref):
    ps = (pa_ref[0] + pa_ref[1])[0:NG, :]
    cnt = (pc_ref[0] + pc_ref[1])[0:NG, 0:1]
    pooled = ps / jnp.maximum(cnt, 1.0)
    mu = s1_ref[...] / N
    var = s2_ref[...] / N - mu * mu
    a = g_ref[...] * lax.rsqrt(var + 1e-5)
    c = b_ref[...] - mu * a
    o_ref[...] = jnp.dot(pooled * a + c, w_ref[...],
                         preferred_element_type=F32)


def _head(pa, pc, s1, s2, g, b, fcwp):
    return pl.pallas_call(
        _head_body,
        out_shape=jax.ShapeDtypeStruct((NG, 128), F32),
    )(pa, pc, s1, s2, g, b, fcwp)


def _sc_edge(y, srcp, dstp, basis_t, bex, cout, cacc, with_count):
    r = 4 * cout
    mesh = plsc.VectorSubcoreMesh(core_axis_name="c", subcore_axis_name="s")

    @functools.partial(
        pl.kernel,
        out_type=jax.ShapeDtypeStruct((NC, NP, cacc), F32),
        mesh=mesh,
        compiler_params=pltpu.CompilerParams(
            needs_layout_passes=False, use_tc_tiling_on_sc=False),
        scratch_types=[
            pltpu.VMEM((BW,), I32),
            pltpu.VMEM((BW,), I32),
            pltpu.VMEM((4, BW) if cout < 16 else (1, 16), F32),
            pltpu.VMEM((BW, 64) if cout >= 16 else (1, 16), F32),
            pltpu.VMEM((BW, r), F32),
            pltpu.VMEM((BW, cacc), F32),
            pltpu.VMEM((ZB, cacc), F32),
            pltpu.VMEM_SHARED((NP, cacc), F32),
            pltpu.SemaphoreType.DMA,
        ],
    )
    def k(y_hbm, src_hbm, dst_hbm, bas_hbm, bex_hbm, out_hbm,
          src_v, dst_v, bas_v, bex_v, rows_v, msg_v, zb_v, acc_sh, sem):
        cid = lax.axis_index("c")
        sid = lax.axis_index("s")
        wid = sid * NC + cid
        iot = lax.iota(I32, 16)
        zvec = jnp.zeros((16,), F32)
        zrow = jnp.zeros((16,), I32)

        for i in range(ZB):
            for c0 in range(0, cacc, 16):
                zb_v[i, pl.ds(c0, 16)] = zvec
        r0 = sid * NPS

        def zbody(w, carry):
            pltpu.sync_copy(zb_v, acc_sh.at[pl.ds(r0 + w * ZB, ZB), :])
            return carry
        lax.fori_loop(0, ZW, zbody, 0)
        plsc.subcore_barrier()

        if with_count:
            cntvec = (iot == 8).astype(F32)

            def ibody(i, carry):
                plsc.store_scatter(msg_v, [zrow + i, iot], cntvec)
                return carry
            lax.fori_loop(0, BW, ibody, 0)

        e_base = wid * EW

        def wbody(w, carry):
            e0 = e_base + w * BW
            pltpu.sync_copy(src_hbm.at[pl.ds(e0, BW)], src_v)
            pltpu.sync_copy(dst_hbm.at[pl.ds(e0, BW)], dst_v)
            if cout < 16:
                for kk in range(4):
                    pltpu.sync_copy(bas_hbm.at[kk, pl.ds(e0, BW)],
                                    bas_v.at[kk])
            else:
                pltpu.sync_copy(bex_hbm.at[pl.ds(e0, BW), :], bex_v)
            pltpu.async_copy(y_hbm.at[src_v], rows_v, sem).wait()

            if cout < 16:
                @plsc.parallel_loop(0, BW // 16, unroll=4)
                def jbody(j):
                    erow = j * 16 + iot
                    bs = [bas_v[kk, pl.ds(j * 16, 16)] for kk in range(4)]
                    for c in range(cout):
                        acc = bs[0] * plsc.load_gather(
                            rows_v, [erow, zrow + c])
                        for kk in range(1, 4):
                            acc = acc + bs[kk] * plsc.load_gather(
                                rows_v, [erow, zrow + (kk * cout + c)])
                        plsc.store_scatter(msg_v, [erow, zrow + c], acc)
            else:
                nh = cout // 16

                @plsc.parallel_loop(0, BW, unroll=4)
                def bbody(b):
                    bs = [bex_v[b, pl.ds(16 * kk, 16)] for kk in range(4)]
                    for h in range(nh):
                        acc = bs[0] * rows_v[b, pl.ds(16 * h, 16)]
                        for kk in range(1, 4):
                            acc = acc + bs[kk] * rows_v[
                                b, pl.ds(kk * cout + 16 * h, 16)]
                        msg_v[b, pl.ds(16 * h, 16)] = acc
            pltpu.sync_copy(msg_v, acc_sh.at[dst_v], add=True)
            return carry
        lax.fori_loop(0, NWIN, wbody, 0)
        plsc.subcore_barrier()

        def obody(w, carry):
            sl = pl.ds(r0 + w * ZB, ZB)
            pltpu.sync_copy(acc_sh.at[sl, :], out_hbm.at[cid, sl, :])
            return carry
        lax.fori_loop(0, ZW, obody, 0)

    return k(y, srcp, dstp, basis_t, bex)


def _sc_pool(t3, bidp):
    mesh = plsc.VectorSubcoreMesh(core_axis_name="c", subcore_axis_name="s")

    @functools.partial(
        pl.kernel,
        out_type=(
            jax.ShapeDtypeStruct((NC, NG + 1, 32), F32),
            jax.ShapeDtypeStruct((NC, NG + 1, 16), F32),
        ),
        mesh=mesh,
        compiler_params=pltpu.CompilerParams(
            needs_layout_passes=False, use_tc_tiling_on_sc=False),
        scratch_types=[
            pltpu.VMEM((BP, 32), F32),
            pltpu.VMEM((BP, 16), F32),
            pltpu.VMEM((BP,), I32),
            pltpu.VMEM_SHARED((NG + 1, 32), F32),
            pltpu.VMEM_SHARED((NG + 1, 16), F32),
        ],
    )
    def k(t3_hbm, bid_hbm, pa_hbm, pc_hbm,
          tw_v, ones_v, bid_v, pacc_sh, pcnt_sh):
        cid = lax.axis_index("c")
        sid = lax.axis_index("s")
        wid = sid * NC + cid
        iot = lax.iota(I32, 16)
        zvec = jnp.zeros((16,), F32)
        zrow = jnp.zeros((16,), I32)
        ovec = jnp.ones((16,), F32)

        def zb(i, carry):
            plsc.store_scatter(tw_v, [zrow + i, iot], zvec)
            plsc.store_scatter(tw_v, [zrow + i, iot + 16], zvec)
            plsc.store_scatter(ones_v, [zrow + i, iot], ovec)
            return carry
        lax.fori_loop(0, BP, zb, 0)

        @pl.when(sid == 0)
        def _():
            pltpu.sync_copy(tw_v.at[pl.ds(0, NG + 1), :], pacc_sh)
            pltpu.sync_copy(tw_v.at[pl.ds(0, NG + 1), 0:16], pcnt_sh)
        plsc.subcore_barrier()

        n_base = wid * NPW

        def wbody(w, carry):
            n0 = n_base + w * BP
            pltpu.sync_copy(t3_hbm.at[pl.ds(n0, BP), :], tw_v)
            pltpu.sync_copy(bid_hbm.at[pl.ds(n0, BP)], bid_v)
            pltpu.sync_copy(tw_v, pacc_sh.at[bid_v], add=True)
            pltpu.sync_copy(ones_v, pcnt_sh.at[bid_v], add=True)
            return carry
        lax.fori_loop(0, NPWIN, wbody, 0)
        plsc.subcore_barrier()

        @pl.when(sid == 0)
        def _():
            pltpu.sync_copy(pacc_sh, pa_hbm.at[cid])
            pltpu.sync_copy(pcnt_sh, pc_hbm.at[cid])

    return k(t3, bidp)


def _wcat(w):
    return jnp.transpose(w, (1, 0, 2)).reshape(w.shape[1], 4 * w.shape[2])


@jax.jit
def kernel(x, edge_index, edge_attr, batch,
           W1, g1, b1, W2, g2, b2, W3, g3, b3, fcW):
    src = edge_index[0]
    dst = edge_index[1]
    srcp = jnp.pad(src, (0, EP - E))
    dstp = jnp.pad(dst, (0, EP - E), constant_values=N)
    eat = jnp.pad(edge_attr, ((0, EP - E), (0, 0))).T
    eat = jnp.pad(eat, ((0, 6), (0, 0)))
    eap = jnp.pad(edge_attr, ((0, EP - E), (0, 6)))
    xp = jnp.pad(x, ((0, NP - N), (0, 5)))
    bidp = jnp.pad(batch, (0, NP - N), constant_values=NG)

    w1c = jnp.pad(_wcat(W1), ((0, 5), (0, 0)))
    w2c = jnp.pad(_wcat(W2), ((0, 8), (0, 0)))
    w3c = _wcat(W3)
    g1p = jnp.pad(g1, (0, 8)).reshape(1, 16)
    b1p = jnp.pad(b1, (0, 8)).reshape(1, 16)
    g2r = g2.reshape(1, 16)
    b2r = b2.reshape(1, 16)
    g3r = g3.reshape(1, 32)
    b3r = b3.reshape(1, 32)
    fcwp = jnp.pad(fcW, ((0, 0), (0, 118)))

    basis_t = _basis(eat)
    bex = _bex(eap)
    y1 = _mm_plain(xp, w1c)
    acc1 = _sc_edge(y1, srcp, dstp, basis_t, bex, cout=8, cacc=16,
                    with_count=True)
    t1, inv16, s1a, s2a = _fin1(acc1)
    y2 = _mm_bn(t1, s1a, s2a, g1p, b1p, w2c)
    acc2 = _sc_edge(y2, srcp, dstp, basis_t, bex, cout=16, cacc=16,
                    with_count=False)
    t2, s1b, s2b = _fin23(acc2, inv16, 16)
    y3 = _mm_bn(t2, s1b, s2b, g2r, b2r, w3c)
    acc3 = _sc_edge(y3, srcp, dstp, basis_t, bex, cout=32, cacc=32,
                    with_count=False)
    t3, s1c, s2c = _fin23(acc3, inv16, 32)
    pa, pc = _sc_pool(t3, bidp)
    out = _head(pa, pc, s1c, s2c, g3r, b3r, fcwp)
    return out[:, :10]


# speedup vs baseline: 1.5558x; 1.1855x over previous
import functools

import jax
import jax.numpy as jnp
from jax import lax
from jax.experimental import pallas as pl
from jax.experimental.pallas import tpu as pltpu
from jax.experimental.pallas import tpu_sc as plsc

N = 50000
NP = 50176
E = 800000
EP = 819200
NG = 64
NC = 2
NS = 16
NW = NC * NS
EW = EP // NW
BW = 128
NWIN = EW // BW
BN_BLK = 1024
GRID_N = NP // BN_BLK
BE_BLK = 6400
GRID_E = EP // BE_BLK
ZB = 32
NPS = NP // NS
ZW = NPS // ZB
NPW = NP // NW
BP = 112
NPWIN = NPW // BP
F32 = jnp.float32
I32 = jnp.int32


def _elu(x):
    return jnp.where(x > 0, x, jnp.exp(jnp.where(x > 0, 0.0, x)) - 1.0)


BE2 = 2048
GRID_E2 = EP // BE2


def _bex_body(e_ref, o_ref):
    f0 = e_ref[:, 0:1]
    f1 = e_ref[:, 1:2]
    parts = []
    for b in [(1.0 - f0) * (1.0 - f1), f0 * (1.0 - f1),
              (1.0 - f0) * f1, f0 * f1]:
        parts.append(jnp.broadcast_to(b, (BE2, 16)))
    o_ref[...] = jnp.concatenate(parts, axis=1)


def _bex(eap):
    return pl.pallas_call(
        _bex_body,
        grid=(GRID_E2,),
        in_specs=[pl.BlockSpec((BE2, 8), lambda i: (i, 0))],
        out_specs=pl.BlockSpec((BE2, 64), lambda i: (i, 0)),
        out_shape=jax.ShapeDtypeStruct((EP, 64), F32),
    )(eap)


def _mm_body(x_ref, w_ref, o_ref):
    o_ref[...] = jnp.dot(x_ref[...], w_ref[...], preferred_element_type=F32)


def _mm_plain(xp, wc):
    cin, r = wc.shape
    return pl.pallas_call(
        _mm_body,
        grid=(GRID_N,),
        in_specs=[
            pl.BlockSpec((BN_BLK, cin), lambda i: (i, 0)),
            pl.BlockSpec((cin, r), lambda i: (0, 0)),
        ],
        out_specs=pl.BlockSpec((BN_BLK, r), lambda i: (i, 0)),
        out_shape=jax.ShapeDtypeStruct((NP, r), F32),
    )(xp, wc)


def _mm_bn_body(t_ref, s1_ref, s2_ref, g_ref, b_ref, w_ref, o_ref):
    mu = s1_ref[...] / N
    var = s2_ref[...] / N - mu * mu
    a = g_ref[...] * lax.rsqrt(var + 1e-5)
    c = b_ref[...] - mu * a
    o_ref[...] = jnp.dot(t_ref[...] * a + c, w_ref[...],
                         preferred_element_type=F32)


def _mm_bn(t, s1, s2, g, b, wc):
    cin, r = wc.shape
    return pl.pallas_call(
        _mm_bn_body,
        grid=(GRID_N,),
        in_specs=[
            pl.BlockSpec((BN_BLK, cin), lambda i: (i, 0)),
            pl.BlockSpec((1, cin), lambda i: (0, 0)),
            pl.BlockSpec((1, cin), lambda i: (0, 0)),
            pl.BlockSpec((1, cin), lambda i: (0, 0)),
            pl.BlockSpec((1, cin), lambda i: (0, 0)),
            pl.BlockSpec((cin, r), lambda i: (0, 0)),
        ],
        out_specs=pl.BlockSpec((BN_BLK, r), lambda i: (i, 0)),
        out_shape=jax.ShapeDtypeStruct((NP, r), F32),
    )(t, s1, s2, g, b, wc)


def _fin1_body(acc_ref, t_ref, inv_ref, s1_ref, s2_ref):
    i = pl.program_id(0)
    a = acc_ref[0] + acc_ref[1]
    cnt = a[:, 8:9]
    inv = 1.0 / jnp.maximum(cnt, 1.0)
    rows = i * BN_BLK + lax.broadcasted_iota(I32, (BN_BLK, 16), 0)
    cols = lax.broadcasted_iota(I32, (BN_BLK, 16), 1)
    t = _elu(a * inv)
    t = jnp.where((rows < N) & (cols < 8), t, 0.0)
    t_ref[...] = t
    inv_ref[...] = jnp.broadcast_to(inv, (BN_BLK, 16))

    @pl.when(i == 0)
    def _():
        s1_ref[...] = jnp.zeros_like(s1_ref)
        s2_ref[...] = jnp.zeros_like(s2_ref)

    s1_ref[...] += jnp.sum(t, axis=0, keepdims=True)
    s2_ref[...] += jnp.sum(t * t, axis=0, keepdims=True)


def _fin1(acc):
    return pl.pallas_call(
        _fin1_body,
        grid=(GRID_N,),
        in_specs=[pl.BlockSpec((2, BN_BLK, 16), lambda i: (0, i, 0))],
        out_specs=[
            pl.BlockSpec((BN_BLK, 16), lambda i: (i, 0)),
            pl.BlockSpec((BN_BLK, 16), lambda i: (i, 0)),
            pl.BlockSpec((1, 16), lambda i: (0, 0)),
            pl.BlockSpec((1, 16), lambda i: (0, 0)),
        ],
        out_shape=[
            jax.ShapeDtypeStruct((NP, 16), F32),
            jax.ShapeDtypeStruct((NP, 16), F32),
            jax.ShapeDtypeStruct((1, 16), F32),
            jax.ShapeDtypeStruct((1, 16), F32),
        ],
    )(acc)


def _fin23_body(c_dim, acc_ref, inv_ref, t_ref, s1_ref, s2_ref):
    i = pl.program_id(0)
    a = acc_ref[0] + acc_ref[1]
    inv = inv_ref[:, 0:1]
    rows = i * BN_BLK + lax.broadcasted_iota(I32, (BN_BLK, c_dim), 0)
    t = _elu(a * inv)
    t = jnp.where(rows < N, t, 0.0)
    t_ref[...] = t

    @pl.when(i == 0)
    def _():
        s1_ref[...] = jnp.zeros_like(s1_ref)
        s2_ref[...] = jnp.zeros_like(s2_ref)

    s1_ref[...] += jnp.sum(t, axis=0, keepdims=True)
    s2_ref[...] += jnp.sum(t * t, axis=0, keepdims=True)


def _fin23(acc, inv16, c_dim):
    return pl.pallas_call(
        functools.partial(_fin23_body, c_dim),
        grid=(GRID_N,),
        in_specs=[
            pl.BlockSpec((2, BN_BLK, c_dim), lambda i: (0, i, 0)),
            pl.BlockSpec((BN_BLK, 16), lambda i: (i, 0)),
        ],
        out_specs=[
            pl.BlockSpec((BN_BLK, c_dim), lambda i: (i, 0)),
            pl.BlockSpec((1, c_dim), lambda i: (0, 0)),
            pl.BlockSpec((1, c_dim), lambda i: (0, 0)),
        ],
        out_shape=[
            jax.ShapeDtypeStruct((NP, c_dim), F32),
            jax.ShapeDtypeStruct((1, c_dim), F32),
            jax.ShapeDtypeStruct((1, c_dim), F32),
        ],
    )(acc, inv16)


def _head_body(pa_ref, pc_ref, s1_ref, s2_ref, g_ref, b_ref, w_ref, o_ref):
    ps = (pa_ref[0] + pa_ref[1])[0:NG, :]
    cnt = (pc_ref[0] + pc_ref[1])[0:NG, 0:1]
    pooled = ps / jnp.maximum(cnt, 1.0)
    mu = s1_ref[...] / N
    var = s2_ref[...] / N - mu * mu
    a = g_ref[...] * lax.rsqrt(var + 1e-5)
    c = b_ref[...] - mu * a
    o_ref[...] = jnp.dot(pooled * a + c, w_ref[...],
                         preferred_element_type=F32)


def _head(pa, pc, s1, s2, g, b, fcwp):
    return pl.pallas_call(
        _head_body,
        out_shape=jax.ShapeDtypeStruct((NG, 128), F32),
    )(pa, pc, s1, s2, g, b, fcwp)


def _sc_edge(y, srcp, dstp, bex, cout, cacc, with_count):
    gw = max(cout, 16)
    nh = gw // 16
    r = 4 * gw
    mesh = plsc.VectorSubcoreMesh(core_axis_name="c", subcore_axis_name="s")

    @functools.partial(
        pl.kernel,
        out_type=jax.ShapeDtypeStruct((NC, NP, cacc), F32),
        mesh=mesh,
        compiler_params=pltpu.CompilerParams(
            needs_layout_passes=False, use_tc_tiling_on_sc=False),
        scratch_types=[
            pltpu.VMEM((BW,), I32),
            pltpu.VMEM((BW,), I32),
            pltpu.VMEM((BW, 64), F32),
            pltpu.VMEM((BW, r), F32),
            pltpu.VMEM((BW, cacc), F32),
            pltpu.VMEM((ZB, cacc), F32),
            pltpu.VMEM_SHARED((NP, cacc), F32),
            pltpu.SemaphoreType.DMA,
        ],
    )
    def k(y_hbm, src_hbm, dst_hbm, bex_hbm, out_hbm,
          src_v, dst_v, bex_v, rows_v, msg_v, zb_v, acc_sh, sem):
        cid = lax.axis_index("c")
        sid = lax.axis_index("s")
        wid = sid * NC + cid
        iot = lax.iota(I32, 16)
        zvec = jnp.zeros((16,), F32)

        for i in range(ZB):
            for c0 in range(0, cacc, 16):
                zb_v[i, pl.ds(c0, 16)] = zvec
        r0 = sid * NPS

        def zbody(w, carry):
            pltpu.sync_copy(zb_v, acc_sh.at[pl.ds(r0 + w * ZB, ZB), :])
            return carry
        lax.fori_loop(0, ZW, zbody, 0)
        plsc.subcore_barrier()

        e_base = wid * EW
        cntvec = (iot == 8).astype(F32)

        def wbody(w, carry):
            e0 = e_base + w * BW
            pltpu.sync_copy(src_hbm.at[pl.ds(e0, BW)], src_v)
            gd = pltpu.async_copy(y_hbm.at[src_v], rows_v, sem)
            pltpu.sync_copy(dst_hbm.at[pl.ds(e0, BW)], dst_v)
            pltpu.sync_copy(bex_hbm.at[pl.ds(e0, BW), :], bex_v)
            gd.wait()

            @plsc.parallel_loop(0, BW, unroll=4)
            def bbody(b):
                bs = [bex_v[b, pl.ds(16 * kk, 16)] for kk in range(4)]
                for h in range(nh):
                    acc = bs[0] * rows_v[b, pl.ds(16 * h, 16)]
                    for kk in range(1, 4):
                        acc = acc + bs[kk] * rows_v[
                            b, pl.ds(kk * gw + 16 * h, 16)]
                    if with_count and h == 0:
                        acc = acc + cntvec
                    msg_v[b, pl.ds(16 * h, 16)] = acc
            pltpu.sync_copy(msg_v, acc_sh.at[dst_v], add=True)
            return carry
        lax.fori_loop(0, NWIN, wbody, 0)
        plsc.subcore_barrier()

        def obody(w, carry):
            sl = pl.ds(r0 + w * ZB, ZB)
            pltpu.sync_copy(acc_sh.at[sl, :], out_hbm.at[cid, sl, :])
            return carry
        lax.fori_loop(0, ZW, obody, 0)

    return k(y, srcp, dstp, bex)


def _sc_pool(t3, bidp):
    mesh = plsc.VectorSubcoreMesh(core_axis_name="c", subcore_axis_name="s")

    @functools.partial(
        pl.kernel,
        out_type=(
            jax.ShapeDtypeStruct((NC, NG + 1, 32), F32),
            jax.ShapeDtypeStruct((NC, NG + 1, 16), F32),
        ),
        mesh=mesh,
        compiler_params=pltpu.CompilerParams(
            needs_layout_passes=False, use_tc_tiling_on_sc=False),
        scratch_types=[
            pltpu.VMEM((BP, 32), F32),
            pltpu.VMEM((BP, 16), F32),
            pltpu.VMEM((BP,), I32),
            pltpu.VMEM_SHARED((NG + 1, 32), F32),
            pltpu.VMEM_SHARED((NG + 1, 16), F32),
        ],
    )
    def k(t3_hbm, bid_hbm, pa_hbm, pc_hbm,
          tw_v, ones_v, bid_v, pacc_sh, pcnt_sh):
        cid = lax.axis_index("c")
        sid = lax.axis_index("s")
        wid = sid * NC + cid
        iot = lax.iota(I32, 16)
        zvec = jnp.zeros((16,), F32)
        zrow = jnp.zeros((16,), I32)
        ovec = jnp.ones((16,), F32)

        def zb(i, carry):
            plsc.store_scatter(tw_v, [zrow + i, iot], zvec)
            plsc.store_scatter(tw_v, [zrow + i, iot + 16], zvec)
            plsc.store_scatter(ones_v, [zrow + i, iot], ovec)
            return carry
        lax.fori_loop(0, BP, zb, 0)

        @pl.when(sid == 0)
        def _():
            pltpu.sync_copy(tw_v.at[pl.ds(0, NG + 1), :], pacc_sh)
            pltpu.sync_copy(tw_v.at[pl.ds(0, NG + 1), 0:16], pcnt_sh)
        plsc.subcore_barrier()

        n_base = wid * NPW

        def wbody(w, carry):
            n0 = n_base + w * BP
            pltpu.sync_copy(t3_hbm.at[pl.ds(n0, BP), :], tw_v)
            pltpu.sync_copy(bid_hbm.at[pl.ds(n0, BP)], bid_v)
            pltpu.sync_copy(tw_v, pacc_sh.at[bid_v], add=True)
            pltpu.sync_copy(ones_v, pcnt_sh.at[bid_v], add=True)
            return carry
        lax.fori_loop(0, NPWIN, wbody, 0)
        plsc.subcore_barrier()

        @pl.when(sid == 0)
        def _():
            pltpu.sync_copy(pacc_sh, pa_hbm.at[cid])
            pltpu.sync_copy(pcnt_sh, pc_hbm.at[cid])

    return k(t3, bidp)


def _wcat(w):
    return jnp.transpose(w, (1, 0, 2)).reshape(w.shape[1], 4 * w.shape[2])


@jax.jit
def kernel(x, edge_index, edge_attr, batch,
           W1, g1, b1, W2, g2, b2, W3, g3, b3, fcW):
    src = edge_index[0]
    dst = edge_index[1]
    srcp = jnp.pad(src, (0, EP - E))
    dstp = jnp.pad(dst, (0, EP - E), constant_values=N)
    eap = jnp.pad(edge_attr, ((0, EP - E), (0, 6)))
    xp = jnp.pad(x, ((0, NP - N), (0, 5)))
    bidp = jnp.pad(batch, (0, NP - N), constant_values=NG)

    w1c = jnp.pad(_wcat(W1).reshape(3, 4, 8), ((0, 5), (0, 0), (0, 8)))
    w1c = w1c.reshape(8, 64)
    w2c = jnp.pad(_wcat(W2), ((0, 8), (0, 0)))
    w3c = _wcat(W3)
    g1p = jnp.pad(g1, (0, 8)).reshape(1, 16)
    b1p = jnp.pad(b1, (0, 8)).reshape(1, 16)
    g2r = g2.reshape(1, 16)
    b2r = b2.reshape(1, 16)
    g3r = g3.reshape(1, 32)
    b3r = b3.reshape(1, 32)
    fcwp = jnp.pad(fcW, ((0, 0), (0, 118)))

    bex = _bex(eap)
    y1 = _mm_plain(xp, w1c)
    acc1 = _sc_edge(y1, srcp, dstp, bex, cout=8, cacc=16,
                    with_count=True)
    t1, inv16, s1a, s2a = _fin1(acc1)
    y2 = _mm_bn(t1, s1a, s2a, g1p, b1p, w2c)
    acc2 = _sc_edge(y2, srcp, dstp, bex, cout=16, cacc=16,
                    with_count=False)
    t2, s1b, s2b = _fin23(acc2, inv16, 16)
    y3 = _mm_bn(t2, s1b, s2b, g2r, b2r, w3c)
    acc3 = _sc_edge(y3, srcp, dstp, bex, cout=32, cacc=32,
                    with_count=False)
    t3, s1c, s2c = _fin23(acc3, inv16, 32)
    pa, pc = _sc_pool(t3, bidp)
    out = _head(pa, pc, s1c, s2c, g3r, b3r, fcwp)
    return out[:, :10]


# speedup vs baseline: 1.5577x; 1.0012x over previous
import functools

import jax
import jax.numpy as jnp
from jax import lax
from jax.experimental import pallas as pl
from jax.experimental.pallas import tpu as pltpu
from jax.experimental.pallas import tpu_sc as plsc

N = 50000
NP = 50176
E = 800000
EP = 819200
NG = 64
NC = 2
NS = 16
NW = NC * NS
EW = EP // NW
BW = 128
NWIN = EW // BW
BN_BLK = 1024
GRID_N = NP // BN_BLK
BE_BLK = 6400
GRID_E = EP // BE_BLK
ZB = 32
NPS = NP // NS
ZW = NPS // ZB
NPW = NP // NW
BP = 112
NPWIN = NPW // BP
F32 = jnp.float32
I32 = jnp.int32


def _elu(x):
    return jnp.where(x > 0, x, jnp.exp(jnp.where(x > 0, 0.0, x)) - 1.0)


BE2 = 2048
GRID_E2 = EP // BE2


def _bex_body(e_ref, o_ref):
    f0 = e_ref[:, 0:1]
    f1 = e_ref[:, 1:2]
    parts = []
    for b in [(1.0 - f0) * (1.0 - f1), f0 * (1.0 - f1),
              (1.0 - f0) * f1, f0 * f1]:
        parts.append(jnp.broadcast_to(b, (BE2, 16)))
    o_ref[...] = jnp.concatenate(parts, axis=1)


def _bex(eap):
    return pl.pallas_call(
        _bex_body,
        grid=(GRID_E2,),
        in_specs=[pl.BlockSpec((BE2, 8), lambda i: (i, 0))],
        out_specs=pl.BlockSpec((BE2, 64), lambda i: (i, 0)),
        out_shape=jax.ShapeDtypeStruct((EP, 64), F32),
    )(eap)


def _mm_body(x_ref, w_ref, o_ref):
    o_ref[...] = jnp.dot(x_ref[...], w_ref[...], preferred_element_type=F32)


def _mm_plain(xp, wc):
    cin, r = wc.shape
    return pl.pallas_call(
        _mm_body,
        grid=(GRID_N,),
        in_specs=[
            pl.BlockSpec((BN_BLK, cin), lambda i: (i, 0)),
            pl.BlockSpec((cin, r), lambda i: (0, 0)),
        ],
        out_specs=pl.BlockSpec((BN_BLK, r), lambda i: (i, 0)),
        out_shape=jax.ShapeDtypeStruct((NP, r), F32),
    )(xp, wc)


def _mm_bn_body(t_ref, s1_ref, s2_ref, g_ref, b_ref, w_ref, o_ref):
    mu = s1_ref[...] / N
    var = s2_ref[...] / N - mu * mu
    a = g_ref[...] * lax.rsqrt(var + 1e-5)
    c = b_ref[...] - mu * a
    o_ref[...] = jnp.dot(t_ref[...] * a + c, w_ref[...],
                         preferred_element_type=F32)


def _mm_bn(t, s1, s2, g, b, wc):
    cin, r = wc.shape
    return pl.pallas_call(
        _mm_bn_body,
        grid=(GRID_N,),
        in_specs=[
            pl.BlockSpec((BN_BLK, cin), lambda i: (i, 0)),
            pl.BlockSpec((1, cin), lambda i: (0, 0)),
            pl.BlockSpec((1, cin), lambda i: (0, 0)),
            pl.BlockSpec((1, cin), lambda i: (0, 0)),
            pl.BlockSpec((1, cin), lambda i: (0, 0)),
            pl.BlockSpec((cin, r), lambda i: (0, 0)),
        ],
        out_specs=pl.BlockSpec((BN_BLK, r), lambda i: (i, 0)),
        out_shape=jax.ShapeDtypeStruct((NP, r), F32),
    )(t, s1, s2, g, b, wc)


def _fin1_body(acc_ref, t_ref, inv_ref, s1_ref, s2_ref):
    i = pl.program_id(0)
    a = acc_ref[0] + acc_ref[1]
    cnt = a[:, 8:9]
    inv = 1.0 / jnp.maximum(cnt, 1.0)
    rows = i * BN_BLK + lax.broadcasted_iota(I32, (BN_BLK, 16), 0)
    cols = lax.broadcasted_iota(I32, (BN_BLK, 16), 1)
    t = _elu(a * inv)
    t = jnp.where((rows < N) & (cols < 8), t, 0.0)
    t_ref[...] = t
    inv_ref[...] = jnp.broadcast_to(inv, (BN_BLK, 16))

    @pl.when(i == 0)
    def _():
        s1_ref[...] = jnp.zeros_like(s1_ref)
        s2_ref[...] = jnp.zeros_like(s2_ref)

    s1_ref[...] += jnp.sum(t, axis=0, keepdims=True)
    s2_ref[...] += jnp.sum(t * t, axis=0, keepdims=True)


def _fin1(acc):
    return pl.pallas_call(
        _fin1_body,
        grid=(GRID_N,),
        in_specs=[pl.BlockSpec((2, BN_BLK, 16), lambda i: (0, i, 0))],
        out_specs=[
            pl.BlockSpec((BN_BLK, 16), lambda i: (i, 0)),
            pl.BlockSpec((BN_BLK, 16), lambda i: (i, 0)),
            pl.BlockSpec((1, 16), lambda i: (0, 0)),
            pl.BlockSpec((1, 16), lambda i: (0, 0)),
        ],
        out_shape=[
            jax.ShapeDtypeStruct((NP, 16), F32),
            jax.ShapeDtypeStruct((NP, 16), F32),
            jax.ShapeDtypeStruct((1, 16), F32),
            jax.ShapeDtypeStruct((1, 16), F32),
        ],
    )(acc)


def _fin23_body(c_dim, acc_ref, inv_ref, t_ref, s1_ref, s2_ref):
    i = pl.program_id(0)
    a = acc_ref[0] + acc_ref[1]
    inv = inv_ref[:, 0:1]
    rows = i * BN_BLK + lax.broadcasted_iota(I32, (BN_BLK, c_dim), 0)
    t = _elu(a * inv)
    t = jnp.where(rows < N, t, 0.0)
    t_ref[...] = t

    @pl.when(i == 0)
    def _():
        s1_ref[...] = jnp.zeros_like(s1_ref)
        s2_ref[...] = jnp.zeros_like(s2_ref)

    s1_ref[...] += jnp.sum(t, axis=0, keepdims=True)
    s2_ref[...] += jnp.sum(t * t, axis=0, keepdims=True)


def _fin23(acc, inv16, c_dim):
    return pl.pallas_call(
        functools.partial(_fin23_body, c_dim),
        grid=(GRID_N,),
        in_specs=[
            pl.BlockSpec((2, BN_BLK, c_dim), lambda i: (0, i, 0)),
            pl.BlockSpec((BN_BLK, 16), lambda i: (i, 0)),
        ],
        out_specs=[
            pl.BlockSpec((BN_BLK, c_dim), lambda i: (i, 0)),
            pl.BlockSpec((1, c_dim), lambda i: (0, 0)),
            pl.BlockSpec((1, c_dim), lambda i: (0, 0)),
        ],
        out_shape=[
            jax.ShapeDtypeStruct((NP, c_dim), F32),
            jax.ShapeDtypeStruct((1, c_dim), F32),
            jax.ShapeDtypeStruct((1, c_dim), F32),
        ],
    )(acc, inv16)


def _head_body(pa_ref, pc_ref, s1_ref, s2_ref, g_ref, b_ref, w_ref, o_ref):
    ps = (pa_ref[0] + pa_ref[1])[0:NG, :]
    cnt = (pc_ref[0] + pc_ref[1])[0:NG, 0:1]
    pooled = ps / jnp.maximum(cnt, 1.0)
    mu = s1_ref[...] / N
    var = s2_ref[...] / N - mu * mu
    a = g_ref[...] * lax.rsqrt(var + 1e-5)
    c = b_ref[...] - mu * a
    o_ref[...] = jnp.dot(pooled * a + c, w_ref[...],
                         preferred_element_type=F32)


def _head(pa, pc, s1, s2, g, b, fcwp):
    return pl.pallas_call(
        _head_body,
        out_shape=jax.ShapeDtypeStruct((NG, 128), F32),
    )(pa, pc, s1, s2, g, b, fcwp)


def _sc_edge(y, srcp, dstp, bex, cout, cacc, with_count):
    gw = max(cout, 16)
    nh = gw // 16
    r = 4 * gw
    mesh = plsc.VectorSubcoreMesh(core_axis_name="c", subcore_axis_name="s")

    @functools.partial(
        pl.kernel,
        out_type=jax.ShapeDtypeStruct((NC, NP, cacc), F32),
        mesh=mesh,
        compiler_params=pltpu.CompilerParams(
            needs_layout_passes=False, use_tc_tiling_on_sc=False),
        scratch_types=[
            pltpu.VMEM((BW,), I32),
            pltpu.VMEM((BW,), I32),
            pltpu.VMEM((BW, 64), F32),
            pltpu.VMEM((BW, r), F32),
            pltpu.VMEM((BW, cacc), F32),
            pltpu.VMEM((ZB, cacc), F32),
            pltpu.VMEM_SHARED((NP, cacc), F32),
            pltpu.SemaphoreType.DMA,
        ],
    )
    def k(y_hbm, src_hbm, dst_hbm, bex_hbm, out_hbm,
          src_v, dst_v, bex_v, rows_v, msg_v, zb_v, acc_sh, sem):
        cid = lax.axis_index("c")
        sid = lax.axis_index("s")
        wid = sid * NC + cid
        iot = lax.iota(I32, 16)
        zvec = jnp.zeros((16,), F32)

        for i in range(ZB):
            for c0 in range(0, cacc, 16):
                zb_v[i, pl.ds(c0, 16)] = zvec
        r0 = sid * NPS

        def zbody(w, carry):
            pltpu.sync_copy(zb_v, acc_sh.at[pl.ds(r0 + w * ZB, ZB), :])
            return carry
        lax.fori_loop(0, ZW, zbody, 0)
        plsc.subcore_barrier()

        e_base = wid * EW
        cntvec = (iot == 8).astype(F32)

        def wbody(w, carry):
            e0 = e_base + w * BW
            pltpu.sync_copy(src_hbm.at[pl.ds(e0, BW)], src_v)
            gd = pltpu.async_copy(y_hbm.at[src_v], rows_v, sem)
            pltpu.sync_copy(dst_hbm.at[pl.ds(e0, BW)], dst_v)
            pltpu.sync_copy(bex_hbm.at[pl.ds(e0, BW), :], bex_v)
            gd.wait()

            @plsc.parallel_loop(0, BW, unroll=8)
            def bbody(b):
                bs = [bex_v[b, pl.ds(16 * kk, 16)] for kk in range(4)]
                for h in range(nh):
                    acc = bs[0] * rows_v[b, pl.ds(16 * h, 16)]
                    for kk in range(1, 4):
                        acc = acc + bs[kk] * rows_v[
                            b, pl.ds(kk * gw + 16 * h, 16)]
                    if with_count and h == 0:
                        acc = acc + cntvec
                    msg_v[b, pl.ds(16 * h, 16)] = acc
            pltpu.sync_copy(msg_v, acc_sh.at[dst_v], add=True)
            return carry
        lax.fori_loop(0, NWIN, wbody, 0)
        plsc.subcore_barrier()

        def obody(w, carry):
            sl = pl.ds(r0 + w * ZB, ZB)
            pltpu.sync_copy(acc_sh.at[sl, :], out_hbm.at[cid, sl, :])
            return carry
        lax.fori_loop(0, ZW, obody, 0)

    return k(y, srcp, dstp, bex)


def _sc_pool(t3, bidp):
    mesh = plsc.VectorSubcoreMesh(core_axis_name="c", subcore_axis_name="s")

    @functools.partial(
        pl.kernel,
        out_type=(
            jax.ShapeDtypeStruct((NC, NG + 1, 32), F32),
            jax.ShapeDtypeStruct((NC, NG + 1, 16), F32),
        ),
        mesh=mesh,
        compiler_params=pltpu.CompilerParams(
            needs_layout_passes=False, use_tc_tiling_on_sc=False),
        scratch_types=[
            pltpu.VMEM((BP, 32), F32),
            pltpu.VMEM((BP, 16), F32),
            pltpu.VMEM((BP,), I32),
            pltpu.VMEM_SHARED((NG + 1, 32), F32),
            pltpu.VMEM_SHARED((NG + 1, 16), F32),
        ],
    )
    def k(t3_hbm, bid_hbm, pa_hbm, pc_hbm,
          tw_v, ones_v, bid_v, pacc_sh, pcnt_sh):
        cid = lax.axis_index("c")
        sid = lax.axis_index("s")
        wid = sid * NC + cid
        iot = lax.iota(I32, 16)
        zvec = jnp.zeros((16,), F32)
        zrow = jnp.zeros((16,), I32)
        ovec = jnp.ones((16,), F32)

        def zb(i, carry):
            plsc.store_scatter(tw_v, [zrow + i, iot], zvec)
            plsc.store_scatter(tw_v, [zrow + i, iot + 16], zvec)
            plsc.store_scatter(ones_v, [zrow + i, iot], ovec)
            return carry
        lax.fori_loop(0, BP, zb, 0)

        @pl.when(sid == 0)
        def _():
            pltpu.sync_copy(tw_v.at[pl.ds(0, NG + 1), :], pacc_sh)
            pltpu.sync_copy(tw_v.at[pl.ds(0, NG + 1), 0:16], pcnt_sh)
        plsc.subcore_barrier()

        n_base = wid * NPW

        def wbody(w, carry):
            n0 = n_base + w * BP
            pltpu.sync_copy(t3_hbm.at[pl.ds(n0, BP), :], tw_v)
            pltpu.sync_copy(bid_hbm.at[pl.ds(n0, BP)], bid_v)
            pltpu.sync_copy(tw_v, pacc_sh.at[bid_v], add=True)
            pltpu.sync_copy(ones_v, pcnt_sh.at[bid_v], add=True)
            return carry
        lax.fori_loop(0, NPWIN, wbody, 0)
        plsc.subcore_barrier()

        @pl.when(sid == 0)
        def _():
            pltpu.sync_copy(pacc_sh, pa_hbm.at[cid])
            pltpu.sync_copy(pcnt_sh, pc_hbm.at[cid])

    return k(t3, bidp)


def _wcat(w):
    return jnp.transpose(w, (1, 0, 2)).reshape(w.shape[1], 4 * w.shape[2])


@jax.jit
def kernel(x, edge_index, edge_attr, batch,
           W1, g1, b1, W2, g2, b2, W3, g3, b3, fcW):
    src = edge_index[0]
    dst = edge_index[1]
    srcp = jnp.pad(src, (0, EP - E))
    dstp = jnp.pad(dst, (0, EP - E), constant_values=N)
    eap = jnp.pad(edge_attr, ((0, EP - E), (0, 6)))
    xp = jnp.pad(x, ((0, NP - N), (0, 5)))
    bidp = jnp.pad(batch, (0, NP - N), constant_values=NG)

    w1c = jnp.pad(_wcat(W1).reshape(3, 4, 8), ((0, 5), (0, 0), (0, 8)))
    w1c = w1c.reshape(8, 64)
    w2c = jnp.pad(_wcat(W2), ((0, 8), (0, 0)))
    w3c = _wcat(W3)
    g1p = jnp.pad(g1, (0, 8)).reshape(1, 16)
    b1p = jnp.pad(b1, (0, 8)).reshape(1, 16)
    g2r = g2.reshape(1, 16)
    b2r = b2.reshape(1, 16)
    g3r = g3.reshape(1, 32)
    b3r = b3.reshape(1, 32)
    fcwp = jnp.pad(fcW, ((0, 0), (0, 118)))

    bex = _bex(eap)
    y1 = _mm_plain(xp, w1c)
    acc1 = _sc_edge(y1, srcp, dstp, bex, cout=8, cacc=16,
                    with_count=True)
    t1, inv16, s1a, s2a = _fin1(acc1)
    y2 = _mm_bn(t1, s1a, s2a, g1p, b1p, w2c)
    acc2 = _sc_edge(y2, srcp, dstp, bex, cout=16, cacc=16,
                    with_count=False)
    t2, s1b, s2b = _fin23(acc2, inv16, 16)
    y3 = _mm_bn(t2, s1b, s2b, g2r, b2r, w3c)
    acc3 = _sc_edge(y3, srcp, dstp, bex, cout=32, cacc=32,
                    with_count=False)
    t3, s1c, s2c = _fin23(acc3, inv16, 32)
    pa, pc = _sc_pool(t3, bidp)
    out = _head(pa, pc, s1c, s2c, g3r, b3r, fcwp)
    return out[:, :10]
